# Initial kernel scaffold; baseline (speedup 1.0000x reference)
import jax
import jax.numpy as jnp
from jax.experimental import pallas as pl


def kernel(x, batch_indices, W_att, b_att, W_score, b_score):
    raise NotImplementedError("write your pallas kernel here")


# speedup vs baseline: 12.7659x; 12.7659x over previous
import functools

import jax
import jax.numpy as jnp
from jax.experimental import pallas as pl
from jax.experimental.pallas import tpu as pltpu


def _flash_body(x_ref, seg_ref, watt_ref, batt_ref, wscore_ref, out_ref,
                m_ref, d_ref, acc_ref, *, num_blocks: int, num_segments: int):
    i = pl.program_id(0)

    @pl.when(i == 0)
    def _init():
        m_ref[...] = jnp.full(m_ref.shape, -jnp.inf, jnp.float32)
        d_ref[...] = jnp.zeros(d_ref.shape, jnp.float32)
        acc_ref[...] = jnp.zeros(acc_ref.shape, jnp.float32)

    xb = x_ref[...]
    feats = jnp.dot(xb, watt_ref[...].T, preferred_element_type=jnp.float32)
    feats = jnp.maximum(feats + batt_ref[...], 0.0)
    scores = jnp.dot(feats, wscore_ref[...],
                     preferred_element_type=jnp.float32)

    blk = xb.shape[0]
    seg = seg_ref[...]
    ids = jax.lax.broadcasted_iota(jnp.int32, (blk, num_segments), 1)
    mask = seg == ids

    m_old = m_ref[...]
    neg_inf = jnp.float32(-jnp.inf)
    m_blk = jnp.max(jnp.where(mask, scores, neg_inf), axis=0, keepdims=True)
    m_new = jnp.maximum(m_old, m_blk)
    corr = jnp.where(m_old == neg_inf, 0.0, jnp.exp(m_old - m_new))

    m_row = jnp.sum(jnp.where(mask, m_new, 0.0), axis=1, keepdims=True)
    ex = jnp.exp(scores - m_row)

    maskf = mask.astype(jnp.float32)
    d_new = d_ref[...] * corr + jnp.sum(maskf * ex, axis=0, keepdims=True)
    contrib_t = jax.lax.dot_general(xb * ex, maskf,
                                    (((0,), (0,)), ((), ())),
                                    preferred_element_type=jnp.float32)
    acc_new = acc_ref[...] * corr + contrib_t

    m_ref[...] = m_new
    d_ref[...] = d_new
    acc_ref[...] = acc_new

    @pl.when(i == num_blocks - 1)
    def _finish():
        d = d_ref[...]
        safe = jnp.where(d > 0.0, d, 1.0)
        out_ref[...] = jnp.where(d > 0.0, acc_ref[...] / safe, 0.0)


def kernel(x, batch_indices, W_att, b_att, W_score, b_score):
    N, H = x.shape
    B = 256
    BLK = 2560
    num_blocks = pl.cdiv(N, BLK)
    pad = num_blocks * BLK - N
    seg = batch_indices.astype(jnp.int32)
    if pad:
        x = jnp.pad(x, ((0, pad), (0, 0)))
        seg = jnp.pad(seg, (0, pad), constant_values=B)
    seg_col = seg.reshape(num_blocks * BLK, 1)

    grid = (num_blocks,)
    out_t = pl.pallas_call(
        functools.partial(_flash_body, num_blocks=num_blocks, num_segments=B),
        grid=grid,
        in_specs=[
            pl.BlockSpec((BLK, H), lambda i: (i, 0)),
            pl.BlockSpec((BLK, 1), lambda i: (i, 0)),
            pl.BlockSpec((H, H), lambda i: (0, 0)),
            pl.BlockSpec((1, H), lambda i: (0, 0)),
            pl.BlockSpec((H, 1), lambda i: (0, 0)),
        ],
        out_specs=pl.BlockSpec((H, B), lambda i: (0, 0)),
        out_shape=jax.ShapeDtypeStruct((H, B), jnp.float32),
        scratch_shapes=[
            pltpu.VMEM((1, B), jnp.float32),
            pltpu.VMEM((1, B), jnp.float32),
            pltpu.VMEM((H, B), jnp.float32),
        ],
    )(x, seg_col, W_att, b_att.reshape(1, H), W_score.reshape(H, 1))
    return out_t.T
